# Initial kernel scaffold; baseline (speedup 1.0000x reference)
import jax
import jax.numpy as jnp
from jax.experimental import pallas as pl


def kernel(x, edge_index, edge_weight, params):
    raise NotImplementedError("write your pallas kernel here")


# speedup vs baseline: 30.8110x; 30.8110x over previous
import functools

import jax
import jax.numpy as jnp
from jax import lax
from jax.experimental import pallas as pl
from jax.experimental.pallas import tpu as pltpu
from jax.experimental.pallas import tpu_sc as plsc

T = 4
N = 10000
E = 160000
F_IN = 64
H = 64
LF = 32
GR = 16
HOR = 4
ALPHA = 0.9
BETA = 0.8

NB = 2000
GRID = N // NB

NC = 2
NS = 16
NW = NC * NS
CHUNK = 128
E_PAD = 163840
EPW = E_PAD // NW
NCHUNK = EPW // CHUNK
N_PAD = 10240
RPS = N_PAD // NS
PK = H
PKW = 128

f32 = jnp.float32


def _dot(a, b):
    return lax.dot_general(a, b, (((a.ndim - 1,), (0,)), ((), ())),
                           preferred_element_type=f32)


def _full(shape):
    return pl.BlockSpec(shape, lambda i: (0,) * len(shape))


def _rows(shape):
    if len(shape) == 2:
        return pl.BlockSpec(shape, lambda i: (i, 0))
    if len(shape) == 3:
        return pl.BlockSpec(shape, lambda i: (0, i, 0))
    raise ValueError(shape)


def _ew_mlp(edge_weight, w1, b1, w2, b2):
    ew2 = edge_weight.reshape(1250, 128)

    def body(ew_ref, w1_ref, b1_ref, w2_ref, b2_ref, o_ref):
        ew = ew_ref[...]
        acc = jnp.zeros_like(ew)
        for j in range(32):
            acc = acc + jnp.maximum(ew * w1_ref[j] + b1_ref[j], 0.0) * w2_ref[j]
        o_ref[...] = acc + b2_ref[0]

    smem = pl.BlockSpec(memory_space=pltpu.SMEM)
    out = pl.pallas_call(
        body,
        grid=(1,),
        in_specs=[pl.BlockSpec((1250, 128), lambda i: (0, 0)),
                  smem, smem, smem, smem],
        out_specs=pl.BlockSpec((1250, 128), lambda i: (0, 0)),
        out_shape=jax.ShapeDtypeStruct((1250, 128), f32),
    )(ew2, w1.reshape(32), b1, w2.reshape(32), b2)
    return out.reshape(E)


def _sc_message_pass(y, src_p, dst_p, ew_p):
    mesh = plsc.VectorSubcoreMesh(core_axis_name="c", subcore_axis_name="s",
                                  num_cores=NC, num_subcores=NS)

    @functools.partial(
        pl.kernel,
        out_type=jax.ShapeDtypeStruct((NC, N_PAD, PKW), f32),
        mesh=mesh,
        scratch_types=[
            pltpu.VMEM_SHARED((N_PAD, PKW), f32),
            pltpu.VMEM((CHUNK,), jnp.int32),
            pltpu.VMEM((CHUNK,), jnp.int32),
            pltpu.VMEM((CHUNK,), f32),
            pltpu.VMEM((CHUNK, PKW), f32),
            pltpu.VMEM((CHUNK, PKW), f32),
            pltpu.SemaphoreType.DMA,
        ],
    )
    def k(y_hbm, src_hbm, dst_hbm, ew_hbm, out_hbm,
          acc_sh, src_v, dst_v, ew_v, rows_v, zero_v, sem):
        cid = lax.axis_index("c")
        sid = lax.axis_index("s")

        def zfill(i, carry):
            for c in range(PKW // 16):
                zero_v[i, pl.ds(c * 16, 16)] = jnp.zeros((16,), f32)
            return carry
        lax.fori_loop(0, CHUNK, zfill, 0)

        row0 = sid * RPS

        def zcopy(j, carry):
            pltpu.sync_copy(zero_v, acc_sh.at[pl.ds(row0 + j * CHUNK, CHUNK)])
            return carry
        lax.fori_loop(0, RPS // CHUNK, zcopy, 0)
        plsc.subcore_barrier()

        ebase = (cid * NS + sid) * EPW

        def chunk_body(g, carry):
            base = ebase + g * CHUNK
            pltpu.sync_copy(src_hbm.at[pl.ds(base, CHUNK)], src_v)
            pltpu.sync_copy(dst_hbm.at[pl.ds(base, CHUNK)], dst_v)
            pltpu.sync_copy(ew_hbm.at[pl.ds(base, CHUNK)], ew_v)
            pltpu.async_copy(y_hbm.at[src_v], rows_v, sem).wait()

            def ebody(g16, c2):
                ewvec = ew_v[pl.ds(g16 * 16, 16)]
                for j in range(16):
                    w = ewvec[j]
                    e = g16 * 16 + j
                    for c in range(PK // 16):
                        sl = pl.ds(c * 16, 16)
                        rows_v[e, sl] = rows_v[e, sl] * w
                return c2
            lax.fori_loop(0, CHUNK // 16, ebody, 0)

            pltpu.sync_copy(rows_v, acc_sh.at[dst_v], add=True)
            return carry
        lax.fori_loop(0, NCHUNK, chunk_body, 0)
        plsc.subcore_barrier()

        def ocopy(j, carry):
            r = row0 + j * CHUNK
            pltpu.sync_copy(acc_sh.at[pl.ds(r, CHUNK)], rows_v)
            pltpu.sync_copy(rows_v, out_hbm.at[cid, pl.ds(r, CHUNK)])
            return carry
        lax.fori_loop(0, RPS // CHUNK, ocopy, 0)

    return k(y, src_p, dst_p, ew_p)


def _k1(x3, node_emb, enc_W, enc_b, skip_W, skip_b, lw0, c0W_out, c0W_lw):
    def body(x_ref, emb_ref, encW_ref, encb_ref, sW_ref, sb_ref,
             lw_ref, cWo_ref, cWl_ref, out_ref, y_ref):
        emb = emb_ref[...]
        lwt = _dot(lw_ref[...], cWl_ref[...])
        for t in range(T):
            h = _dot(x_ref[t], encW_ref[...]) + encb_ref[...] + emb
            o = _dot(h, sW_ref[...]) + sb_ref[...]
            out_ref[t] = o
            y_ref[:, t * GR:(t + 1) * GR] = _dot(o, cWo_ref[...]) + lwt
        y_ref[:, PK:] = jnp.zeros((NB, PKW - PK), f32)

    return pl.pallas_call(
        body,
        grid=(GRID,),
        in_specs=[_rows((T, NB, H)), _rows((NB, H)), _full((F_IN, H)),
                  _full((1, H)), _full((H, H)), _full((1, H)),
                  _rows((NB, LF)), _full((H, GR)), _full((LF, GR))],
        out_specs=[_rows((T, NB, H)), _rows((NB, PKW))],
        out_shape=[jax.ShapeDtypeStruct((T, N, H), f32),
                   jax.ShapeDtypeStruct((N, PKW), f32)],
    )(x3, node_emb, enc_W, enc_b, skip_W, skip_b, lw0, c0W_out, c0W_lw)


def _k2(aggp, out_i, lw, c0b_pk, c1W_out, c1W_lw, c1W_h):
    def body(a_ref, o_ref, lw_ref, c0b_ref, cWo_ref, cWl_ref, cWh_ref, y_ref):
        hl0 = jnp.maximum(a_ref[0, :, :PK] + a_ref[1, :, :PK] + c0b_ref[...],
                          0.0)
        lwt = _dot(lw_ref[...], cWl_ref[...])
        for t in range(T):
            sl = slice(t * GR, (t + 1) * GR)
            y_ref[:, sl] = (_dot(o_ref[t], cWo_ref[...]) + lwt
                            + _dot(hl0[:, sl], cWh_ref[...]))
        y_ref[:, PK:] = jnp.zeros((NB, PKW - PK), f32)

    return pl.pallas_call(
        body,
        grid=(GRID,),
        in_specs=[_rows((NC, NB, PKW)), _rows((T, NB, H)), _rows((NB, LF)),
                  _full((1, PK)), _full((H, GR)), _full((LF, GR)),
                  _full((GR, GR))],
        out_specs=_rows((NB, PKW)),
        out_shape=jax.ShapeDtypeStruct((N, PKW), f32),
    )(aggp, out_i, lw, c0b_pk, c1W_out, c1W_lw, c1W_h)


def _k3(a0p, a1p, out0, lw0, c0b_pk, c1b_pk, pW_out, pW_lw, pW_h0, pW_h1,
        p_b, ng, nb_, skip_W, skip_b, lw1, n_c0W_out, n_c0W_lw):
    def body(a0_ref, a1_ref, o0_ref, lw0_ref, c0b_ref, c1b_ref,
             pWo_ref, pWl_ref, pWh0_ref, pWh1_ref, pb_ref, ng_ref, nb_ref,
             sW_ref, sb_ref, lw1_ref, nWo_ref, nWl_ref, out1_ref, y_ref):
        hl0 = jnp.maximum(a0_ref[0, :, :PK] + a0_ref[1, :, :PK]
                          + c0b_ref[...], 0.0)
        hl1 = jnp.maximum(a1_ref[0, :, :PK] + a1_ref[1, :, :PK]
                          + c1b_ref[...], 0.0)
        lwp = _dot(lw0_ref[...], pWl_ref[...])
        lwt = _dot(lw1_ref[...], nWl_ref[...])
        for t in range(T):
            sl = slice(t * GR, (t + 1) * GR)
            xc = (_dot(o0_ref[t], pWo_ref[...]) + lwp
                  + _dot(hl0[:, sl], pWh0_ref[...])
                  + _dot(hl1[:, sl], pWh1_ref[...]) + pb_ref[...])
            mu = jnp.mean(xc, axis=-1, keepdims=True)
            d = xc - mu
            var = jnp.mean(d * d, axis=-1, keepdims=True)
            xc = ng_ref[...] * d * lax.rsqrt(var + 1e-5) + nb_ref[...]
            o1 = _dot(xc, sW_ref[...]) + sb_ref[...] + o0_ref[t]
            out1_ref[t] = o1
            y_ref[:, sl] = _dot(o1, nWo_ref[...]) + lwt
        y_ref[:, PK:] = jnp.zeros((NB, PKW - PK), f32)

    return pl.pallas_call(
        body,
        grid=(GRID,),
        in_specs=[_rows((NC, NB, PKW)), _rows((NC, NB, PKW)),
                  _rows((T, NB, H)),
                  _rows((NB, LF)), _full((1, PK)), _full((1, PK)),
                  _full((H, H)), _full((LF, H)), _full((GR, H)),
                  _full((GR, H)), _full((1, H)), _full((1, H)), _full((1, H)),
                  _full((H, H)), _full((1, H)), _rows((NB, LF)),
                  _full((H, GR)), _full((LF, GR))],
        out_specs=[_rows((T, NB, H)), _rows((NB, PKW))],
        out_shape=[jax.ShapeDtypeStruct((T, N, H), f32),
                   jax.ShapeDtypeStruct((N, PKW), f32)],
    )(a0p, a1p, out0, lw0, c0b_pk, c1b_pk, pW_out, pW_lw, pW_h0, pW_h1,
      p_b, ng, nb_, skip_W, skip_b, lw1, n_c0W_out, n_c0W_lw)


def _lcoef():
    L = [[0.0] * T for _ in range(T)]
    for t in range(T):
        for tau in range(t + 1):
            L[t][tau] = sum(BETA ** (t - s) * ALPHA ** (s - tau)
                            for s in range(tau, t + 1))
    return L


def _k4(a0p, a1p, out1, lw1, c0b_pk, c1b_pk, pW_out, pW_lw, pW_h0, pW_h1,
        p_b, ng, nb_, tWs, tbs, rdW1, rdb1, rdW2, rdb2):
    L = _lcoef()

    def body(a0_ref, a1_ref, o1_ref, lw1_ref, c0b_ref, c1b_ref,
             pWo_ref, pWl_ref, pWh0_ref, pWh1_ref, pb_ref, ng_ref, nb_ref,
             tW0_ref, tW1_ref, tW2_ref, tb0_ref, tb1_ref, tb2_ref,
             rdW1_ref, rdb1_ref, rdW2_ref, rdb2_ref, out_ref):
        hl0 = jnp.maximum(a0_ref[0, :, :PK] + a0_ref[1, :, :PK]
                          + c0b_ref[...], 0.0)
        hl1 = jnp.maximum(a1_ref[0, :, :PK] + a1_ref[1, :, :PK]
                          + c1b_ref[...], 0.0)
        lwp = _dot(lw1_ref[...], pWl_ref[...])
        hs = []
        for t in range(T):
            sl = slice(t * GR, (t + 1) * GR)
            xc = (_dot(o1_ref[t], pWo_ref[...]) + lwp
                  + _dot(hl0[:, sl], pWh0_ref[...])
                  + _dot(hl1[:, sl], pWh1_ref[...]) + pb_ref[...])
            mu = jnp.mean(xc, axis=-1, keepdims=True)
            d = xc - mu
            var = jnp.mean(d * d, axis=-1, keepdims=True)
            hs.append(ng_ref[...] * d * lax.rsqrt(var + 1e-5) + nb_ref[...])

        tW = [tW0_ref, tW1_ref, tW2_ref]
        tb = [tb0_ref, tb1_ref, tb2_ref]
        for l in range(3):
            inp = [_dot(hs[t], tW[l][...]) + tb[l][...] for t in range(T)]
            if l < 2:
                hs = [sum(L[t][tau] * inp[tau] for tau in range(t + 1))
                      for t in range(T)]
            else:
                last = sum(L[T - 1][tau] * inp[tau] for tau in range(T))

        r = jnp.maximum(last, 0.0)
        r = jnp.maximum(_dot(r, rdW1_ref[...]) + rdb1_ref[...], 0.0)
        r = _dot(r, rdW2_ref[...]) + rdb2_ref[...]
        for hh in range(HOR):
            out_ref[hh] = r[:, hh * F_IN:(hh + 1) * F_IN]

    return pl.pallas_call(
        body,
        grid=(GRID,),
        in_specs=[_rows((NC, NB, PKW)), _rows((NC, NB, PKW)),
                  _rows((T, NB, H)),
                  _rows((NB, LF)), _full((1, PK)), _full((1, PK)),
                  _full((H, H)), _full((LF, H)), _full((GR, H)),
                  _full((GR, H)), _full((1, H)), _full((1, H)), _full((1, H)),
                  _full((H, H)), _full((H, H)), _full((H, H)),
                  _full((1, H)), _full((1, H)), _full((1, H)),
                  _full((H, 4 * H)), _full((1, 4 * H)),
                  _full((4 * H, HOR * F_IN)), _full((1, HOR * F_IN))],
        out_specs=_rows((HOR, NB, F_IN)),
        out_shape=jax.ShapeDtypeStruct((HOR, N, F_IN), f32),
    )(a0p, a1p, out1, lw1, c0b_pk, c1b_pk, pW_out, pW_lw, pW_h0, pW_h1,
      p_b, ng, nb_, tWs[0], tWs[1], tWs[2], tbs[0], tbs[1], tbs[2],
      rdW1, rdb1, rdW2, rdb2)


def kernel(x, edge_index, edge_weight, params):
    p = params
    x3 = x[0]

    ewt = _ew_mlp(edge_weight, p['ew_W1'], p['ew_b1'], p['ew_W2'], p['ew_b2'])
    zpad = jnp.zeros((E_PAD - E,), f32)
    ipad = jnp.zeros((E_PAD - E,), jnp.int32)
    src_p = jnp.concatenate([edge_index[0], ipad])
    dst_p = jnp.concatenate([edge_index[1], ipad])
    ew_p = jnp.concatenate([ewt, zpad])

    def pk_bias(b):
        return jnp.tile(b, T).reshape(1, PK)

    r1 = lambda a: a.reshape(1, -1)

    b0_c0 = p['b0_c0_W']; b0_c1 = p['b0_c1_W']; b0_pj = p['b0_proj_W']
    b1_c0 = p['b1_c0_W']; b1_c1 = p['b1_c1_W']; b1_pj = p['b1_proj_W']

    out0, y0 = _k1(x3, p['node_emb'], p['enc_W'], r1(p['enc_b']),
                   p['b0_skip_W'], r1(p['b0_skip_b']), p['b0_lw'],
                   b0_c0[:H], b0_c0[H:H + LF])
    a00 = _sc_message_pass(y0, src_p, dst_p, ew_p)
    y1 = _k2(a00, out0, p['b0_lw'], pk_bias(p['b0_c0_b']),
             b0_c1[:H], b0_c1[H:H + LF], b0_c1[H + LF:])
    a01 = _sc_message_pass(y1, src_p, dst_p, ew_p)

    out1, y2 = _k3(a00, a01, out0, p['b0_lw'], pk_bias(p['b0_c0_b']),
                   pk_bias(p['b0_c1_b']), b0_pj[:H], b0_pj[H:H + LF],
                   b0_pj[H + LF:H + LF + GR], b0_pj[H + LF + GR:],
                   r1(p['b0_proj_b']), r1(p['b0_ng']), r1(p['b0_nb']),
                   p['b1_skip_W'], r1(p['b1_skip_b']), p['b1_lw'],
                   b1_c0[:H], b1_c0[H:H + LF])
    a10 = _sc_message_pass(y2, src_p, dst_p, ew_p)
    y3 = _k2(a10, out1, p['b1_lw'], pk_bias(p['b1_c0_b']),
             b1_c1[:H], b1_c1[H:H + LF], b1_c1[H + LF:])
    a11 = _sc_message_pass(y3, src_p, dst_p, ew_p)

    out = _k4(a10, a11, out1, p['b1_lw'], pk_bias(p['b1_c0_b']),
              pk_bias(p['b1_c1_b']), b1_pj[:H], b1_pj[H:H + LF],
              b1_pj[H + LF:H + LF + GR], b1_pj[H + LF + GR:],
              r1(p['b1_proj_b']), r1(p['b1_ng']), r1(p['b1_nb']),
              [p['t0_W'], p['t1_W'], p['t2_W']],
              [r1(p['t0_b']), r1(p['t1_b']), r1(p['t2_b'])],
              p['rd_W1'], r1(p['rd_b1']), p['rd_W2'], r1(p['rd_b2']))

    return out[None]


# speedup vs baseline: 41.8379x; 1.3579x over previous
import functools

import jax
import jax.numpy as jnp
from jax import lax
from jax.experimental import pallas as pl
from jax.experimental.pallas import tpu as pltpu
from jax.experimental.pallas import tpu_sc as plsc

T = 4
N = 10000
E = 160000
F_IN = 64
H = 64
LF = 32
GR = 16
HOR = 4
ALPHA = 0.9
BETA = 0.8

NB = 2000
GRID = N // NB

NC = 2
NS = 16
NW = NC * NS
CHUNK = 128
E_PAD = 163840
EPW = E_PAD // NW
NCHUNK = EPW // CHUNK
N_PAD = 10240
RPS = N_PAD // NS
PK = H
PKW = 128

f32 = jnp.float32


def _dot(a, b):
    return lax.dot_general(a, b, (((a.ndim - 1,), (0,)), ((), ())),
                           preferred_element_type=f32)


def _full(shape):
    return pl.BlockSpec(shape, lambda i: (0,) * len(shape))


def _rows(shape):
    if len(shape) == 2:
        return pl.BlockSpec(shape, lambda i: (i, 0))
    if len(shape) == 3:
        return pl.BlockSpec(shape, lambda i: (0, i, 0))
    raise ValueError(shape)


def _ew_mlp(edge_weight, w1, b1, w2, b2):
    ew2 = edge_weight.reshape(1250, 128)

    def body(ew_ref, w1_ref, b1_ref, w2_ref, b2_ref, o_ref):
        ew = ew_ref[...]
        acc = jnp.zeros_like(ew)
        for j in range(32):
            acc = acc + jnp.maximum(ew * w1_ref[j] + b1_ref[j], 0.0) * w2_ref[j]
        o_ref[...] = acc + b2_ref[0]

    smem = pl.BlockSpec(memory_space=pltpu.SMEM)
    out = pl.pallas_call(
        body,
        grid=(1,),
        in_specs=[pl.BlockSpec((1250, 128), lambda i: (0, 0)),
                  smem, smem, smem, smem],
        out_specs=pl.BlockSpec((1250, 128), lambda i: (0, 0)),
        out_shape=jax.ShapeDtypeStruct((1250, 128), f32),
    )(ew2, w1.reshape(32), b1, w2.reshape(32), b2)
    return out.reshape(E)


NBUF = 2


def _sc_message_pass(y, epk, ew_p):
    mesh = plsc.VectorSubcoreMesh(core_axis_name="c", subcore_axis_name="s",
                                  num_cores=NC, num_subcores=NS)

    @functools.partial(
        pl.kernel,
        out_type=jax.ShapeDtypeStruct((NC, N_PAD, PKW), f32),
        mesh=mesh,
        scratch_types=[
            pltpu.VMEM_SHARED((N_PAD, PKW), f32),
            pltpu.VMEM((2 * NCHUNK, CHUNK), jnp.int32),
            pltpu.VMEM((EPW,), f32),
            pltpu.VMEM((NBUF, CHUNK, PKW), f32),
            pltpu.SemaphoreType.DMA,
        ],
    )
    def k(y_hbm, epk_hbm, ew_hbm, out_hbm,
          acc_sh, idx_v, ew_v, rows_v, gsem):
        cid = lax.axis_index("c")
        sid = lax.axis_index("s")
        wid = cid * NS + sid

        pltpu.sync_copy(epk_hbm.at[pl.ds(wid * 2 * NCHUNK, 2 * NCHUNK)],
                        idx_v)
        pltpu.sync_copy(ew_hbm.at[pl.ds(wid * EPW, EPW)], ew_v)

        def zfill(i, carry):
            for c in range(PKW // 16):
                rows_v[0, i, pl.ds(c * 16, 16)] = jnp.zeros((16,), f32)
            return carry
        lax.fori_loop(0, CHUNK, zfill, 0)

        row0 = sid * RPS

        def zcopy(j, carry):
            pltpu.sync_copy(rows_v.at[0],
                            acc_sh.at[pl.ds(row0 + j * CHUNK, CHUNK)])
            return carry
        lax.fori_loop(0, RPS // CHUNK, zcopy, 0)
        plsc.subcore_barrier()

        def gather(g, b):
            return pltpu.make_async_copy(y_hbm.at[idx_v.at[2 * g]],
                                         rows_v.at[b], gsem)

        for b in range(NBUF):
            gather(b, b).start()

        def outer(go, carry):
            for b in range(NBUF):
                g = go * NBUF + b
                gather(g, b).wait()

                def ebody(g16, c2):
                    ewvec = ew_v[pl.ds(g * CHUNK + g16 * 16, 16)]
                    for j in range(16):
                        w = ewvec[j]
                        e = g16 * 16 + j
                        for c in range(PK // 16):
                            sl = pl.ds(c * 16, 16)
                            rows_v[b, e, sl] = rows_v[b, e, sl] * w
                    return c2
                lax.fori_loop(0, CHUNK // 16, ebody, 0)

                pltpu.sync_copy(rows_v.at[b], acc_sh.at[idx_v.at[2 * g + 1]],
                                add=True)

                @pl.when(g + NBUF < NCHUNK)
                def _():
                    gather(g + NBUF, b).start()
            return carry
        lax.fori_loop(0, NCHUNK // NBUF, outer, 0)
        plsc.subcore_barrier()

        def ocopy(j, carry):
            r = row0 + j * CHUNK
            pltpu.sync_copy(acc_sh.at[pl.ds(r, CHUNK)], rows_v.at[0])
            pltpu.sync_copy(rows_v.at[0], out_hbm.at[cid, pl.ds(r, CHUNK)])
            return carry
        lax.fori_loop(0, RPS // CHUNK, ocopy, 0)

    return k(y, epk, ew_p)


def _k1(x3, node_emb, enc_W, enc_b, skip_W, skip_b, lw0, c0W_out, c0W_lw):
    def body(x_ref, emb_ref, encW_ref, encb_ref, sW_ref, sb_ref,
             lw_ref, cWo_ref, cWl_ref, out_ref, y_ref):
        emb = emb_ref[...]
        lwt = _dot(lw_ref[...], cWl_ref[...])
        for t in range(T):
            h = _dot(x_ref[t], encW_ref[...]) + encb_ref[...] + emb
            o = _dot(h, sW_ref[...]) + sb_ref[...]
            out_ref[t] = o
            y_ref[:, t * GR:(t + 1) * GR] = _dot(o, cWo_ref[...]) + lwt
        y_ref[:, PK:] = jnp.zeros((NB, PKW - PK), f32)

    return pl.pallas_call(
        body,
        grid=(GRID,),
        in_specs=[_rows((T, NB, H)), _rows((NB, H)), _full((F_IN, H)),
                  _full((1, H)), _full((H, H)), _full((1, H)),
                  _rows((NB, LF)), _full((H, GR)), _full((LF, GR))],
        out_specs=[_rows((T, NB, H)), _rows((NB, PKW))],
        out_shape=[jax.ShapeDtypeStruct((T, N, H), f32),
                   jax.ShapeDtypeStruct((N, PKW), f32)],
    )(x3, node_emb, enc_W, enc_b, skip_W, skip_b, lw0, c0W_out, c0W_lw)


def _k2(aggp, out_i, lw, c0b_pk, c1W_out, c1W_lw, c1W_h):
    def body(a_ref, o_ref, lw_ref, c0b_ref, cWo_ref, cWl_ref, cWh_ref, y_ref):
        hl0 = jnp.maximum(a_ref[0, :, :PK] + a_ref[1, :, :PK] + c0b_ref[...],
                          0.0)
        lwt = _dot(lw_ref[...], cWl_ref[...])
        for t in range(T):
            sl = slice(t * GR, (t + 1) * GR)
            y_ref[:, sl] = (_dot(o_ref[t], cWo_ref[...]) + lwt
                            + _dot(hl0[:, sl], cWh_ref[...]))
        y_ref[:, PK:] = jnp.zeros((NB, PKW - PK), f32)

    return pl.pallas_call(
        body,
        grid=(GRID,),
        in_specs=[_rows((NC, NB, PKW)), _rows((T, NB, H)), _rows((NB, LF)),
                  _full((1, PK)), _full((H, GR)), _full((LF, GR)),
                  _full((GR, GR))],
        out_specs=_rows((NB, PKW)),
        out_shape=jax.ShapeDtypeStruct((N, PKW), f32),
    )(aggp, out_i, lw, c0b_pk, c1W_out, c1W_lw, c1W_h)


def _k3(a0p, a1p, out0, lw0, c0b_pk, c1b_pk, pW_out, pW_lw, pW_h0, pW_h1,
        p_b, ng, nb_, skip_W, skip_b, lw1, n_c0W_out, n_c0W_lw):
    def body(a0_ref, a1_ref, o0_ref, lw0_ref, c0b_ref, c1b_ref,
             pWo_ref, pWl_ref, pWh0_ref, pWh1_ref, pb_ref, ng_ref, nb_ref,
             sW_ref, sb_ref, lw1_ref, nWo_ref, nWl_ref, out1_ref, y_ref):
        hl0 = jnp.maximum(a0_ref[0, :, :PK] + a0_ref[1, :, :PK]
                          + c0b_ref[...], 0.0)
        hl1 = jnp.maximum(a1_ref[0, :, :PK] + a1_ref[1, :, :PK]
                          + c1b_ref[...], 0.0)
        lwp = _dot(lw0_ref[...], pWl_ref[...])
        lwt = _dot(lw1_ref[...], nWl_ref[...])
        for t in range(T):
            sl = slice(t * GR, (t + 1) * GR)
            xc = (_dot(o0_ref[t], pWo_ref[...]) + lwp
                  + _dot(hl0[:, sl], pWh0_ref[...])
                  + _dot(hl1[:, sl], pWh1_ref[...]) + pb_ref[...])
            mu = jnp.mean(xc, axis=-1, keepdims=True)
            d = xc - mu
            var = jnp.mean(d * d, axis=-1, keepdims=True)
            xc = ng_ref[...] * d * lax.rsqrt(var + 1e-5) + nb_ref[...]
            o1 = _dot(xc, sW_ref[...]) + sb_ref[...] + o0_ref[t]
            out1_ref[t] = o1
            y_ref[:, sl] = _dot(o1, nWo_ref[...]) + lwt
        y_ref[:, PK:] = jnp.zeros((NB, PKW - PK), f32)

    return pl.pallas_call(
        body,
        grid=(GRID,),
        in_specs=[_rows((NC, NB, PKW)), _rows((NC, NB, PKW)),
                  _rows((T, NB, H)),
                  _rows((NB, LF)), _full((1, PK)), _full((1, PK)),
                  _full((H, H)), _full((LF, H)), _full((GR, H)),
                  _full((GR, H)), _full((1, H)), _full((1, H)), _full((1, H)),
                  _full((H, H)), _full((1, H)), _rows((NB, LF)),
                  _full((H, GR)), _full((LF, GR))],
        out_specs=[_rows((T, NB, H)), _rows((NB, PKW))],
        out_shape=[jax.ShapeDtypeStruct((T, N, H), f32),
                   jax.ShapeDtypeStruct((N, PKW), f32)],
    )(a0p, a1p, out0, lw0, c0b_pk, c1b_pk, pW_out, pW_lw, pW_h0, pW_h1,
      p_b, ng, nb_, skip_W, skip_b, lw1, n_c0W_out, n_c0W_lw)


def _lcoef():
    L = [[0.0] * T for _ in range(T)]
    for t in range(T):
        for tau in range(t + 1):
            L[t][tau] = sum(BETA ** (t - s) * ALPHA ** (s - tau)
                            for s in range(tau, t + 1))
    return L


def _k4(a0p, a1p, out1, lw1, c0b_pk, c1b_pk, pW_out, pW_lw, pW_h0, pW_h1,
        p_b, ng, nb_, tWs, tbs, rdW1, rdb1, rdW2, rdb2):
    L = _lcoef()

    def body(a0_ref, a1_ref, o1_ref, lw1_ref, c0b_ref, c1b_ref,
             pWo_ref, pWl_ref, pWh0_ref, pWh1_ref, pb_ref, ng_ref, nb_ref,
             tW0_ref, tW1_ref, tW2_ref, tb0_ref, tb1_ref, tb2_ref,
             rdW1_ref, rdb1_ref, rdW2_ref, rdb2_ref, out_ref):
        hl0 = jnp.maximum(a0_ref[0, :, :PK] + a0_ref[1, :, :PK]
                          + c0b_ref[...], 0.0)
        hl1 = jnp.maximum(a1_ref[0, :, :PK] + a1_ref[1, :, :PK]
                          + c1b_ref[...], 0.0)
        lwp = _dot(lw1_ref[...], pWl_ref[...])
        hs = []
        for t in range(T):
            sl = slice(t * GR, (t + 1) * GR)
            xc = (_dot(o1_ref[t], pWo_ref[...]) + lwp
                  + _dot(hl0[:, sl], pWh0_ref[...])
                  + _dot(hl1[:, sl], pWh1_ref[...]) + pb_ref[...])
            mu = jnp.mean(xc, axis=-1, keepdims=True)
            d = xc - mu
            var = jnp.mean(d * d, axis=-1, keepdims=True)
            hs.append(ng_ref[...] * d * lax.rsqrt(var + 1e-5) + nb_ref[...])

        tW = [tW0_ref, tW1_ref, tW2_ref]
        tb = [tb0_ref, tb1_ref, tb2_ref]
        for l in range(3):
            inp = [_dot(hs[t], tW[l][...]) + tb[l][...] for t in range(T)]
            if l < 2:
                hs = [sum(L[t][tau] * inp[tau] for tau in range(t + 1))
                      for t in range(T)]
            else:
                last = sum(L[T - 1][tau] * inp[tau] for tau in range(T))

        r = jnp.maximum(last, 0.0)
        r = jnp.maximum(_dot(r, rdW1_ref[...]) + rdb1_ref[...], 0.0)
        r = _dot(r, rdW2_ref[...]) + rdb2_ref[...]
        for hh in range(HOR):
            out_ref[hh] = r[:, hh * F_IN:(hh + 1) * F_IN]

    return pl.pallas_call(
        body,
        grid=(GRID,),
        in_specs=[_rows((NC, NB, PKW)), _rows((NC, NB, PKW)),
                  _rows((T, NB, H)),
                  _rows((NB, LF)), _full((1, PK)), _full((1, PK)),
                  _full((H, H)), _full((LF, H)), _full((GR, H)),
                  _full((GR, H)), _full((1, H)), _full((1, H)), _full((1, H)),
                  _full((H, H)), _full((H, H)), _full((H, H)),
                  _full((1, H)), _full((1, H)), _full((1, H)),
                  _full((H, 4 * H)), _full((1, 4 * H)),
                  _full((4 * H, HOR * F_IN)), _full((1, HOR * F_IN))],
        out_specs=_rows((HOR, NB, F_IN)),
        out_shape=jax.ShapeDtypeStruct((HOR, N, F_IN), f32),
    )(a0p, a1p, out1, lw1, c0b_pk, c1b_pk, pW_out, pW_lw, pW_h0, pW_h1,
      p_b, ng, nb_, tWs[0], tWs[1], tWs[2], tbs[0], tbs[1], tbs[2],
      rdW1, rdb1, rdW2, rdb2)


def kernel(x, edge_index, edge_weight, params):
    p = params
    x3 = x[0]

    ewt = _ew_mlp(edge_weight, p['ew_W1'], p['ew_b1'], p['ew_W2'], p['ew_b2'])
    zpad = jnp.zeros((E_PAD - E,), f32)
    ipad = jnp.zeros((E_PAD - E,), jnp.int32)
    src_p = jnp.concatenate([edge_index[0], ipad])
    dst_p = jnp.concatenate([edge_index[1], ipad])
    ew_p = jnp.concatenate([ewt, zpad])
    epk = jnp.stack([src_p.reshape(-1, CHUNK), dst_p.reshape(-1, CHUNK)],
                    axis=1).reshape(-1, CHUNK)

    def pk_bias(b):
        return jnp.tile(b, T).reshape(1, PK)

    r1 = lambda a: a.reshape(1, -1)

    b0_c0 = p['b0_c0_W']; b0_c1 = p['b0_c1_W']; b0_pj = p['b0_proj_W']
    b1_c0 = p['b1_c0_W']; b1_c1 = p['b1_c1_W']; b1_pj = p['b1_proj_W']

    out0, y0 = _k1(x3, p['node_emb'], p['enc_W'], r1(p['enc_b']),
                   p['b0_skip_W'], r1(p['b0_skip_b']), p['b0_lw'],
                   b0_c0[:H], b0_c0[H:H + LF])
    a00 = _sc_message_pass(y0, epk, ew_p)
    y1 = _k2(a00, out0, p['b0_lw'], pk_bias(p['b0_c0_b']),
             b0_c1[:H], b0_c1[H:H + LF], b0_c1[H + LF:])
    a01 = _sc_message_pass(y1, epk, ew_p)

    out1, y2 = _k3(a00, a01, out0, p['b0_lw'], pk_bias(p['b0_c0_b']),
                   pk_bias(p['b0_c1_b']), b0_pj[:H], b0_pj[H:H + LF],
                   b0_pj[H + LF:H + LF + GR], b0_pj[H + LF + GR:],
                   r1(p['b0_proj_b']), r1(p['b0_ng']), r1(p['b0_nb']),
                   p['b1_skip_W'], r1(p['b1_skip_b']), p['b1_lw'],
                   b1_c0[:H], b1_c0[H:H + LF])
    a10 = _sc_message_pass(y2, epk, ew_p)
    y3 = _k2(a10, out1, p['b1_lw'], pk_bias(p['b1_c0_b']),
             b1_c1[:H], b1_c1[H:H + LF], b1_c1[H + LF:])
    a11 = _sc_message_pass(y3, epk, ew_p)

    out = _k4(a10, a11, out1, p['b1_lw'], pk_bias(p['b1_c0_b']),
              pk_bias(p['b1_c1_b']), b1_pj[:H], b1_pj[H:H + LF],
              b1_pj[H + LF:H + LF + GR], b1_pj[H + LF + GR:],
              r1(p['b1_proj_b']), r1(p['b1_ng']), r1(p['b1_nb']),
              [p['t0_W'], p['t1_W'], p['t2_W']],
              [r1(p['t0_b']), r1(p['t1_b']), r1(p['t2_b'])],
              p['rd_W1'], r1(p['rd_b1']), p['rd_W2'], r1(p['rd_b2']))

    return out[None]
